# Initial kernel scaffold; baseline (speedup 1.0000x reference)
import jax
import jax.numpy as jnp
from jax.experimental import pallas as pl


def kernel(x_user, x_item, ei_user_to_item, ei_item_to_user, Wp_user, bp_user, Wp_item, bp_item, Wk0_user, Wq0_user, Wv0_user, Wa0_user, bk0_user, bq0_user, bv0_user, ba0_user, skip0_user, Wk0_item, Wq0_item, Wv0_item, Wa0_item, bk0_item, bq0_item, bv0_item, ba0_item, skip0_item, Watt0_u2i, Wmsg0_u2i, prel0_u2i, Watt0_i2u, Wmsg0_i2u, prel0_i2u, Wk1_user, Wq1_user, Wv1_user, Wa1_user, bk1_user, bq1_user, bv1_user, ba1_user, skip1_user, Wk1_item, Wq1_item, Wv1_item, Wa1_item, bk1_item, bq1_item, bv1_item, ba1_item, skip1_item, Watt1_u2i, Wmsg1_u2i, prel1_u2i, Watt1_i2u, Wmsg1_i2u, prel1_i2u):
    raise NotImplementedError("write your pallas kernel here")


# speedup vs baseline: 1.3671x; 1.3671x over previous
import functools
import numpy as np
import jax
import jax.numpy as jnp
from jax.experimental import pallas as pl

N_NODES = 50000
E_EDGES = 300000
D = 128
H = 4
DH = 32
_INV_SQRT_DH = 1.0 / np.sqrt(32.0)

ROW_BLK = 2000


def _mm_act_body(x_ref, w_ref, b_ref, o_ref, *, act):
    y = jnp.dot(x_ref[...], w_ref[...], preferred_element_type=jnp.float32)
    y = y + b_ref[...]
    if act == "relu":
        y = jnp.maximum(y, 0.0)
    o_ref[...] = y


def _mm_act(x, w, b, act):
    n, d = x.shape
    f = w.shape[1]
    grid = (n // ROW_BLK,)
    return pl.pallas_call(
        functools.partial(_mm_act_body, act=act),
        grid=grid,
        in_specs=[
            pl.BlockSpec((ROW_BLK, d), lambda i: (i, 0)),
            pl.BlockSpec((d, f), lambda i: (0, 0)),
            pl.BlockSpec((1, f), lambda i: (0, 0)),
        ],
        out_specs=pl.BlockSpec((ROW_BLK, f), lambda i: (i, 0)),
        out_shape=jax.ShapeDtypeStruct((n, f), jnp.float32),
    )(x, w, b)


def _update_body(agg_ref, x_ref, wa_ref, ba_ref, skip_ref, o_ref):
    g = jax.nn.gelu(agg_ref[...])
    out = jnp.dot(g, wa_ref[...], preferred_element_type=jnp.float32) + ba_ref[...]
    a = jax.nn.sigmoid(skip_ref[0, 0])
    o_ref[...] = jnp.maximum(a * out + (1.0 - a) * x_ref[...], 0.0)


def _update(agg, x, wa, ba, skip):
    n = x.shape[0]
    return pl.pallas_call(
        _update_body,
        grid=(n // ROW_BLK,),
        in_specs=[
            pl.BlockSpec((ROW_BLK, D), lambda i: (i, 0)),
            pl.BlockSpec((ROW_BLK, D), lambda i: (i, 0)),
            pl.BlockSpec((D, D), lambda i: (0, 0)),
            pl.BlockSpec((1, D), lambda i: (0, 0)),
            pl.BlockSpec((1, 1), lambda i: (0, 0)),
        ],
        out_specs=pl.BlockSpec((ROW_BLK, D), lambda i: (i, 0)),
        out_shape=jax.ShapeDtypeStruct((n, D), jnp.float32),
    )(agg, x, wa, ba, skip)


def _fold_k(Wk, bk, Watt):
    Wk_eff = jnp.einsum("ihd,hdf->ihf", Wk.reshape(D, H, DH), Watt).reshape(D, D)
    bk_eff = jnp.einsum("hd,hdf->hf", bk.reshape(H, DH), Watt).reshape(D)
    return Wk_eff, bk_eff


def _rel_edge(Kt, Vt, Q, ei, prel):
    src = ei[0]
    dst = ei[1]
    ke = Kt[src].reshape(-1, H, DH)
    qe = Q[dst].reshape(-1, H, DH)
    score = (qe * ke).sum(-1) * prel[None, :] * _INV_SQRT_DH
    m = jax.ops.segment_max(score, dst, num_segments=N_NODES)
    ex = jnp.exp(score - m[dst])
    den = jax.ops.segment_sum(ex, dst, num_segments=N_NODES)
    alpha = ex / (den[dst] + 1e-16)
    msg = Vt[src].reshape(-1, H, DH)
    agg = jax.ops.segment_sum(msg * alpha[..., None], dst, num_segments=N_NODES)
    return agg.reshape(N_NODES, D)


def kernel(x_user, x_item, ei_user_to_item, ei_item_to_user, Wp_user, bp_user, Wp_item, bp_item, Wk0_user, Wq0_user, Wv0_user, Wa0_user, bk0_user, bq0_user, bv0_user, ba0_user, skip0_user, Wk0_item, Wq0_item, Wv0_item, Wa0_item, bk0_item, bq0_item, bv0_item, ba0_item, skip0_item, Watt0_u2i, Wmsg0_u2i, prel0_u2i, Watt0_i2u, Wmsg0_i2u, prel0_i2u, Wk1_user, Wq1_user, Wv1_user, Wa1_user, bk1_user, bq1_user, bv1_user, ba1_user, skip1_user, Wk1_item, Wq1_item, Wv1_item, Wa1_item, bk1_item, bq1_item, bv1_item, ba1_item, skip1_item, Watt1_u2i, Wmsg1_u2i, prel1_u2i, Watt1_i2u, Wmsg1_i2u, prel1_i2u):
    fl = dict(locals())

    xu = _mm_act(x_user, Wp_user, bp_user.reshape(1, D), "relu")
    xi = _mm_act(x_item, Wp_item, bp_item.reshape(1, D), "relu")

    for l in range(2):
        Wku, bku = _fold_k(fl[f"Wk{l}_user"], fl[f"bk{l}_user"], fl[f"Watt{l}_u2i"])
        Wvu, bvu = _fold_k(fl[f"Wv{l}_user"], fl[f"bv{l}_user"], fl[f"Wmsg{l}_u2i"])
        Wki, bki = _fold_k(fl[f"Wk{l}_item"], fl[f"bk{l}_item"], fl[f"Watt{l}_i2u"])
        Wvi, bvi = _fold_k(fl[f"Wv{l}_item"], fl[f"bv{l}_item"], fl[f"Wmsg{l}_i2u"])

        wu = jnp.concatenate([Wku, fl[f"Wq{l}_user"], Wvu], axis=1)
        bu = jnp.concatenate([bku, fl[f"bq{l}_user"], bvu]).reshape(1, 3 * D)
        wi = jnp.concatenate([Wki, fl[f"Wq{l}_item"], Wvi], axis=1)
        bi = jnp.concatenate([bki, fl[f"bq{l}_item"], bvi]).reshape(1, 3 * D)

        kqv_u = _mm_act(xu, wu, bu, "none")
        kqv_i = _mm_act(xi, wi, bi, "none")
        Ktu, Qu, Vtu = kqv_u[:, :D], kqv_u[:, D:2 * D], kqv_u[:, 2 * D:]
        Kti, Qi, Vti = kqv_i[:, :D], kqv_i[:, D:2 * D], kqv_i[:, 2 * D:]

        agg_i = _rel_edge(Ktu, Vtu, Qi, ei_user_to_item, fl[f"prel{l}_u2i"])
        agg_u = _rel_edge(Kti, Vti, Qu, ei_item_to_user, fl[f"prel{l}_i2u"])

        xu = _update(agg_u, xu, fl[f"Wa{l}_user"], fl[f"ba{l}_user"].reshape(1, D),
                     fl[f"skip{l}_user"].reshape(1, 1))
        xi = _update(agg_i, xi, fl[f"Wa{l}_item"], fl[f"ba{l}_item"].reshape(1, D),
                     fl[f"skip{l}_item"].reshape(1, 1))

    return jnp.concatenate([xu, xi], axis=0)
